# Initial kernel scaffold; baseline (speedup 1.0000x reference)
import jax
import jax.numpy as jnp
from jax.experimental import pallas as pl


def kernel(set_positions, router_emb):
    raise NotImplementedError("write your pallas kernel here")


# speedup vs baseline: 1.5346x; 1.5346x over previous
import functools

import jax
import jax.numpy as jnp
from jax import lax
from jax.experimental import pallas as pl
from jax.experimental.pallas import tpu as pltpu
from jax.experimental.pallas import tpu_sc as plsc

D_MODEL = 64
B_SETS = 1024
S_LEN = 200
GAMMA = 1.0
BETA = 0.0

N_TOTAL = B_SETS * S_LEN
NUM_CORES = 2
NUM_SUBCORES = 16
NW = NUM_CORES * NUM_SUBCORES
PER_W = N_TOTAL // NW
CHUNK = 128
NCHUNK = PER_W // CHUNK


@functools.partial(
    pl.kernel,
    mesh=plsc.VectorSubcoreMesh(core_axis_name="c", subcore_axis_name="s"),
    out_type=jax.ShapeDtypeStruct((N_TOTAL, D_MODEL), jnp.float32),
    scratch_types=[
        pltpu.VMEM((NCHUNK, CHUNK), jnp.int32),
        pltpu.VMEM((CHUNK, D_MODEL), jnp.float32),
        pltpu.SemaphoreType.DMA,
    ],
    compiler_params=pltpu.CompilerParams(use_tc_tiling_on_sc=False),
)
def _sc_gather(idx_hbm, table_hbm, out_hbm, idx_v, rows_v, sem):
    wid = lax.axis_index("s") * NUM_CORES + lax.axis_index("c")
    pltpu.sync_copy(idx_hbm.at[wid], idx_v)
    out_base = wid * PER_W

    def body(j, carry):
        pltpu.async_copy(table_hbm.at[idx_v.at[j]], rows_v, sem).wait()
        pltpu.sync_copy(rows_v, out_hbm.at[pl.ds(out_base + j * CHUNK, CHUNK)])
        return carry

    lax.fori_loop(0, NCHUNK, body, 0)


_BB = 8


def _bias_body(pos_ref, out_ref):
    p = pos_ref[...]
    d = p[:, :, None] - p[:, None, :]
    out_ref[...] = -GAMMA * jnp.abs(d).astype(jnp.float32) + BETA


_bias = pl.pallas_call(
    _bias_body,
    grid=(B_SETS // _BB,),
    in_specs=[pl.BlockSpec((_BB, S_LEN), lambda i: (i, 0))],
    out_specs=pl.BlockSpec((_BB, S_LEN, S_LEN), lambda i: (i, 0, 0)),
    out_shape=jax.ShapeDtypeStruct((B_SETS, S_LEN, S_LEN), jnp.float32),
)


def kernel(set_positions, router_emb):
    idx2d = set_positions.reshape(NW, NCHUNK, CHUNK).astype(jnp.int32)
    gathered = _sc_gather(idx2d, router_emb)
    desc_router = gathered.reshape(B_SETS, S_LEN, D_MODEL)
    geom_bias = _bias(set_positions)
    return (desc_router, geom_bias)


# speedup vs baseline: 2.5022x; 1.6305x over previous
import functools

import jax
import jax.numpy as jnp
from jax import lax
from jax.experimental import pallas as pl
from jax.experimental.pallas import tpu as pltpu
from jax.experimental.pallas import tpu_sc as plsc

D_MODEL = 64
B_SETS = 1024
S_LEN = 200
GAMMA = 1.0
BETA = 0.0

N_TOTAL = B_SETS * S_LEN
NUM_CORES = 2
NUM_SUBCORES = 16
NW = NUM_CORES * NUM_SUBCORES
PER_W = N_TOTAL // NW
CHUNK = 128
NCHUNK = PER_W // CHUNK


@functools.partial(
    pl.kernel,
    mesh=plsc.VectorSubcoreMesh(core_axis_name="c", subcore_axis_name="s"),
    out_type=jax.ShapeDtypeStruct((N_TOTAL, D_MODEL), jnp.float32),
    scratch_types=[
        pltpu.VMEM((NCHUNK, CHUNK), jnp.int32),
        pltpu.VMEM((CHUNK, D_MODEL), jnp.float32),
        pltpu.SemaphoreType.DMA,
    ],
    compiler_params=pltpu.CompilerParams(use_tc_tiling_on_sc=False),
)
def _sc_gather(idx_hbm, table_hbm, out_hbm, idx_v, rows_v, sem):
    wid = lax.axis_index("s") * NUM_CORES + lax.axis_index("c")
    pltpu.sync_copy(idx_hbm.at[wid], idx_v)
    out_base = wid * PER_W

    def body(j, carry):
        pltpu.async_copy(table_hbm.at[idx_v.at[j]], rows_v, sem).wait()
        pltpu.sync_copy(rows_v, out_hbm.at[pl.ds(out_base + j * CHUNK, CHUNK)])
        return carry

    lax.fori_loop(0, NCHUNK, body, 0)


_BI = 8
_BL = 512


def _bias_body(pi_ref, pall_ref, out_ref):
    pi = pi_ref[...]
    pall = pall_ref[...]
    d = pi[:, None, :] - pall[None, :, :]
    out_ref[...] = -GAMMA * jnp.abs(d).astype(jnp.float32) + BETA


_bias_t = pl.pallas_call(
    _bias_body,
    grid=(S_LEN // _BI, B_SETS // _BL),
    in_specs=[
        pl.BlockSpec((_BI, _BL), lambda i, b: (i, b)),
        pl.BlockSpec((S_LEN, _BL), lambda i, b: (0, b)),
    ],
    out_specs=pl.BlockSpec((_BI, S_LEN, _BL), lambda i, b: (i, 0, b)),
    out_shape=jax.ShapeDtypeStruct((S_LEN, S_LEN, B_SETS), jnp.float32),
)


def kernel(set_positions, router_emb):
    idx2d = set_positions.reshape(NW, NCHUNK, CHUNK).astype(jnp.int32)
    gathered = _sc_gather(idx2d, router_emb)
    desc_router = gathered.reshape(B_SETS, S_LEN, D_MODEL)
    pos_t = set_positions.T.astype(jnp.int32)
    geom_bias = jnp.transpose(_bias_t(pos_t, pos_t), (2, 0, 1))
    return (desc_router, geom_bias)


# speedup vs baseline: 2.7996x; 1.1189x over previous
import functools

import jax
import jax.numpy as jnp
from jax import lax
from jax.experimental import pallas as pl
from jax.experimental.pallas import tpu as pltpu
from jax.experimental.pallas import tpu_sc as plsc

D_MODEL = 64
B_SETS = 1024
S_LEN = 200
GAMMA = 1.0
BETA = 0.0

N_TOTAL = B_SETS * S_LEN
NUM_CORES = 2
NUM_SUBCORES = 16
NW = NUM_CORES * NUM_SUBCORES
SETS_PER_W = B_SETS // NW
HALF = S_LEN // 2
BLK = 2
NBLKW = SETS_PER_W // BLK
CHUNKS_PER_W = SETS_PER_W * 2


@functools.partial(
    pl.kernel,
    mesh=plsc.VectorSubcoreMesh(core_axis_name="c", subcore_axis_name="s"),
    out_type=jax.ShapeDtypeStruct((B_SETS, S_LEN, D_MODEL), jnp.float32),
    scratch_types=[
        pltpu.VMEM((CHUNKS_PER_W, HALF), jnp.int32),
        pltpu.VMEM((2, BLK, S_LEN, D_MODEL), jnp.float32),
        pltpu.SemaphoreType.DMA,
        pltpu.SemaphoreType.DMA,
    ],
    compiler_params=pltpu.CompilerParams(use_tc_tiling_on_sc=False),
)
def _sc_gather(idx_hbm, table_hbm, out_hbm, idx_v, bufs, gsem0, gsem1):
    wid = lax.axis_index("s") * NUM_CORES + lax.axis_index("c")
    pltpu.sync_copy(idx_hbm.at[wid], idx_v)
    set_base = wid * SETS_PER_W
    sems = (gsem0, gsem1)

    def issue(k, p):
        for h in range(4):
            pltpu.async_copy(
                table_hbm.at[idx_v.at[k * 4 + h]],
                bufs.at[p, h // 2, pl.ds((h % 2) * HALF, HALF)],
                sems[p])

    def drain(p):
        for _ in range(4):
            pltpu.make_async_copy(
                table_hbm.at[idx_v.at[0]],
                bufs.at[p, 0, pl.ds(0, HALF)],
                sems[p]).wait()

    def step(k, p, kk):
        drain(p)
        pltpu.sync_copy(bufs.at[p], out_hbm.at[pl.ds(set_base + k * BLK, BLK)])

        @pl.when(kk < NBLKW // 2 - 1)
        def _():
            issue(k + 2, p)

    issue(0, 0)
    issue(1, 1)

    def body(kk, carry):
        step(2 * kk, 0, kk)
        step(2 * kk + 1, 1, kk)
        return carry

    lax.fori_loop(0, NBLKW // 2, body, 0)


_BI = 8
_BL = 512


def _bias_body(pi_ref, pall_ref, out_ref):
    pi = pi_ref[...]
    pall = pall_ref[...]
    d = pi[:, None, :] - pall[None, :, :]
    out_ref[...] = -GAMMA * jnp.abs(d).astype(jnp.float32) + BETA


_bias_t = pl.pallas_call(
    _bias_body,
    grid=(S_LEN // _BI, B_SETS // _BL),
    in_specs=[
        pl.BlockSpec((_BI, _BL), lambda i, b: (i, b)),
        pl.BlockSpec((S_LEN, _BL), lambda i, b: (0, b)),
    ],
    out_specs=pl.BlockSpec((_BI, S_LEN, _BL), lambda i, b: (i, 0, b)),
    out_shape=jax.ShapeDtypeStruct((S_LEN, S_LEN, B_SETS), jnp.float32),
)


def kernel(set_positions, router_emb):
    idx3d = set_positions.reshape(NW, CHUNKS_PER_W, HALF).astype(jnp.int32)
    desc_router = _sc_gather(idx3d, router_emb)
    pos_t = set_positions.T.astype(jnp.int32)
    geom_bias = jnp.transpose(_bias_t(pos_t, pos_t), (2, 0, 1))
    return (desc_router, geom_bias)
